# baseline (speedup 1.0000x reference)
import functools

import jax
import jax.numpy as jnp
from jax import lax
from jax.experimental import pallas as pl
from jax.experimental.pallas import tpu as pltpu
from jax.experimental.pallas import tpu_sc as plsc

_B, _S, _NC, _C = 8, 2048, 4096, 512
_D = 256
_SBLK = 256
_NCBLK = 512
_NSTEP = _S // _SBLK

_NWORK = 32
_ROWS_PER_W = _B * _S // _NWORK
_CH = 128


def _stage1_body(q_ref, rt_ref, rf_ref, w1t_ref, fiota_ref, gt_ref, idx_ref):
    b = pl.program_id(0)
    q = q_ref[0]
    rt = rt_ref[0]
    q0, q1, q2 = q[:, 0:1], q[:, 1:2], q[:, 2:3]
    r0, r1, r2 = rt[0:1, :], rt[1:2, :], rt[2:3, :]
    qm2 = (-2.0 * q).astype(jnp.bfloat16)
    rb = rt.astype(jnp.bfloat16)
    acc = lax.dot_general(qm2, rb, (((1,), (0,)), ((), ())),
                          preferred_element_type=jnp.float32)
    qsq = q0 * q0 + q1 * q1 + q2 * q2
    rsq = r0 * r0 + r1 * r1 + r2 * r2
    val = jnp.maximum((qsq + acc[:, 0:128]) + rsq[:, 0:128], 0.0)
    ci = jnp.zeros(val.shape, jnp.float32)
    for c in range(1, _NC // 128):
        lo = c * 128
        dc = jnp.maximum((qsq + acc[:, lo:lo + 128]) + rsq[:, lo:lo + 128], 0.0)
        lt = dc < val
        val = jnp.minimum(val, dc)
        ci = jnp.where(lt, float(lo), ci)
    liota = fiota_ref[:, 0:128]
    idxf = ci + liota
    m = jnp.min(val, axis=1, keepdims=True)
    fidx = jnp.min(jnp.where(val <= m, idxf, float(_NC)), axis=1)
    idx_ref[0, 0, 0, :] = fidx.astype(jnp.int32) + b * _NC
    gt_ref[0] = lax.dot_general(
        rf_ref[0].astype(jnp.bfloat16), w1t_ref[...].astype(jnp.bfloat16),
        (((0,), (0,)), ((), ())),
        preferred_element_type=jnp.float32).astype(jnp.bfloat16)


def _stage1(query_xyz, ref_xyz_t, ref_feat, w1t, fiota):
    return pl.pallas_call(
        _stage1_body,
        grid=(_B, _NSTEP),
        in_specs=[
            pl.BlockSpec((1, _SBLK, 3), lambda b, i: (b, i, 0)),
            pl.BlockSpec((1, 3, _NC), lambda b, i: (b, 0, 0)),
            pl.BlockSpec((1, _C, _NCBLK), lambda b, i: (b, 0, i)),
            pl.BlockSpec((_C, _D), lambda b, i: (0, 0)),
            pl.BlockSpec((1, _NC), lambda b, i: (0, 0)),
        ],
        out_specs=[
            pl.BlockSpec((1, _NCBLK, _D), lambda b, i: (b, i, 0)),
            pl.BlockSpec((1, 1, 1, _SBLK), lambda b, i: (b, i, 0, 0)),
        ],
        out_shape=[
            jax.ShapeDtypeStruct((_B, _NC, _D), jnp.bfloat16),
            jax.ShapeDtypeStruct((_B, _NSTEP, 1, _SBLK), jnp.int32),
        ],
    )(query_xyz, ref_xyz_t, ref_feat, w1t, fiota)


def _sc_gather(table, idx_flat):
    mesh = plsc.VectorSubcoreMesh(core_axis_name="c", subcore_axis_name="s")

    @functools.partial(
        pl.kernel,
        mesh=mesh,
        out_type=jax.ShapeDtypeStruct((_B * _S, _D // 2), jnp.int32),
        scratch_types=[
            pltpu.VMEM((_CH,), jnp.int32),
            pltpu.VMEM((_CH, _D // 2), jnp.int32),
            pltpu.SemaphoreType.DMA,
        ],
    )
    def gather_kernel(table_hbm, idx_hbm, out_hbm, idx_v, rows_v, sem):
        wid = lax.axis_index("s") * 2 + lax.axis_index("c")
        base = wid * _ROWS_PER_W

        def body(i, carry):
            off = base + i * _CH
            pltpu.sync_copy(idx_hbm.at[pl.ds(off, _CH)], idx_v)
            pltpu.async_copy(table_hbm.at[idx_v], rows_v, sem).wait()
            pltpu.sync_copy(rows_v, out_hbm.at[pl.ds(off, _CH)])
            return carry

        lax.fori_loop(0, _ROWS_PER_W // _CH, body, 0)

    return gather_kernel(table, idx_flat)


def _stage3_body(w2a_ref, w2b_ref, seed_ref, g_ref, b1_ref, b2_ref, out_ref):
    bias = lax.dot_general(
        w2b_ref[...], b1_ref[...], (((1,), (0,)), ((), ())),
        preferred_element_type=jnp.float32) + b2_ref[...]
    a = lax.dot_general(
        w2a_ref[...], seed_ref[0], (((1,), (0,)), ((), ())),
        preferred_element_type=jnp.float32)
    gpart = lax.dot_general(
        w2b_ref[...].astype(jnp.bfloat16), g_ref[0], (((1,), (1,)), ((), ())),
        preferred_element_type=jnp.float32)
    out_ref[0] = a + gpart + bias


def _stage3(w2a, w2b, seed_features, g, b1_2d, b2_2d):
    return pl.pallas_call(
        _stage3_body,
        grid=(_B,),
        in_specs=[
            pl.BlockSpec((_D, _D), lambda b: (0, 0)),
            pl.BlockSpec((_D, _D), lambda b: (0, 0)),
            pl.BlockSpec((1, _D, _S), lambda b: (b, 0, 0)),
            pl.BlockSpec((1, _S, _D), lambda b: (b, 0, 0)),
            pl.BlockSpec((_D, 1), lambda b: (0, 0)),
            pl.BlockSpec((_D, 1), lambda b: (0, 0)),
        ],
        out_specs=pl.BlockSpec((1, _D, _S), lambda b: (b, 0, 0)),
        out_shape=jax.ShapeDtypeStruct((_B, _D, _S), jnp.float32),
    )(w2a, w2b, seed_features, g, b1_2d, b2_2d)


def kernel(query_xyz, ref_xyz, ref_feat, seed_features, W1, b1, W2, b2):
    ref_xyz_t = jnp.transpose(ref_xyz, (0, 2, 1))
    w1t = W1.T
    w2a = W2[:, :_D]
    w2b = W2[:, _D:]

    fiota = jnp.arange(_NC, dtype=jnp.float32).reshape(1, _NC)
    gt, idx = _stage1(query_xyz, ref_xyz_t, ref_feat, w1t, fiota)
    table = lax.bitcast_convert_type(
        gt.reshape(_B * _NC, _D // 2, 2), jnp.int32)
    g = _sc_gather(table, idx.reshape(_B * _S))
    g_bf = lax.bitcast_convert_type(g, jnp.bfloat16).reshape(_B, _S, _D)
    out = _stage3(w2a, w2b, seed_features, g_bf,
                  b1.reshape(_D, 1), b2.reshape(_D, 1))
    return out


# speedup vs baseline: 2.3812x; 2.3812x over previous
import functools

import jax
import jax.numpy as jnp
from jax import lax
from jax.experimental import pallas as pl
from jax.experimental.pallas import tpu as pltpu
from jax.experimental.pallas import tpu_sc as plsc

_B, _S, _NC, _C = 8, 2048, 4096, 512
_D = 256
_SBLK = 256
_NCBLK = 512
_NSTEP = _S // _SBLK

_NWORK = 32
_ROWS_PER_W = _B * _S // _NWORK
_CH = 128


def _stage1_body(q_ref, rt_ref, rf_ref, w1t_ref, fiota_ref, gt_ref, idx_ref):
    b = pl.program_id(0)
    q = q_ref[0]
    rt = rt_ref[0]
    q0, q1, q2 = q[:, 0:1], q[:, 1:2], q[:, 2:3]
    r0, r1, r2 = rt[0:1, :], rt[1:2, :], rt[2:3, :]
    qm2 = (-2.0 * q).astype(jnp.bfloat16)
    rb = rt.astype(jnp.bfloat16)
    acc = lax.dot_general(qm2, rb, (((1,), (0,)), ((), ())),
                          preferred_element_type=jnp.float32)
    qsq = q0 * q0 + q1 * q1 + q2 * q2
    rsq = r0 * r0 + r1 * r1 + r2 * r2
    val = jnp.maximum((qsq + acc[:, 0:128]) + rsq[:, 0:128], 0.0)
    ci = jnp.zeros(val.shape, jnp.float32)
    for c in range(1, _NC // 128):
        lo = c * 128
        dc = jnp.maximum((qsq + acc[:, lo:lo + 128]) + rsq[:, lo:lo + 128], 0.0)
        lt = dc < val
        val = jnp.minimum(val, dc)
        ci = jnp.where(lt, float(lo), ci)
    liota = fiota_ref[:, 0:128]
    idxf = ci + liota
    m = jnp.min(val, axis=1, keepdims=True)
    fidx = jnp.min(jnp.where(val <= m, idxf, float(_NC)), axis=1)
    idx_ref[0, 0, 0, :] = fidx.astype(jnp.int32) + b * _NC
    gt_ref[0] = lax.dot_general(
        rf_ref[0].astype(jnp.bfloat16), w1t_ref[...].astype(jnp.bfloat16),
        (((0,), (0,)), ((), ())),
        preferred_element_type=jnp.float32)


def _stage1(query_xyz, ref_xyz_t, ref_feat, w1t, fiota):
    return pl.pallas_call(
        _stage1_body,
        grid=(_B, _NSTEP),
        in_specs=[
            pl.BlockSpec((1, _SBLK, 3), lambda b, i: (b, i, 0)),
            pl.BlockSpec((1, 3, _NC), lambda b, i: (b, 0, 0)),
            pl.BlockSpec((1, _C, _NCBLK), lambda b, i: (b, 0, i)),
            pl.BlockSpec((_C, _D), lambda b, i: (0, 0)),
            pl.BlockSpec((1, _NC), lambda b, i: (0, 0)),
        ],
        out_specs=[
            pl.BlockSpec((1, _NCBLK, _D), lambda b, i: (b, i, 0)),
            pl.BlockSpec((1, 1, 1, _SBLK), lambda b, i: (b, i, 0, 0)),
        ],
        out_shape=[
            jax.ShapeDtypeStruct((_B, _NC, _D), jnp.float32),
            jax.ShapeDtypeStruct((_B, _NSTEP, 1, _SBLK), jnp.int32),
        ],
    )(query_xyz, ref_xyz_t, ref_feat, w1t, fiota)


def _sc_gather(table, idx_flat):
    mesh = plsc.VectorSubcoreMesh(core_axis_name="c", subcore_axis_name="s")

    @functools.partial(
        pl.kernel,
        mesh=mesh,
        out_type=jax.ShapeDtypeStruct((_B * _S, _D), jnp.float32),
        scratch_types=[
            pltpu.VMEM((_CH,), jnp.int32),
            pltpu.VMEM((_CH,), jnp.int32),
            pltpu.VMEM((_CH, _D), jnp.float32),
            pltpu.VMEM((_CH, _D), jnp.float32),
            pltpu.SemaphoreType.DMA,
            pltpu.SemaphoreType.DMA,
        ],
    )
    def gather_kernel(table_hbm, idx_hbm, out_hbm,
                      idx_v0, idx_v1, rows_v0, rows_v1, sem0, sem1):
        wid = lax.axis_index("s") * 2 + lax.axis_index("c")
        base = wid * _ROWS_PER_W
        bufs = ((idx_v0, rows_v0, sem0), (idx_v1, rows_v1, sem1))
        nch = _ROWS_PER_W // _CH
        copies = [None, None]
        for i in range(nch):
            iv, rv, sm = bufs[i % 2]
            pltpu.sync_copy(idx_hbm.at[pl.ds(base + i * _CH, _CH)], iv)
            copies[i % 2] = pltpu.async_copy(table_hbm.at[iv], rv, sm)
            if i >= 1:
                pv, prv, psm = bufs[(i - 1) % 2]
                copies[(i - 1) % 2].wait()
                pltpu.sync_copy(prv, out_hbm.at[pl.ds(base + (i - 1) * _CH, _CH)])
        copies[(nch - 1) % 2].wait()
        pltpu.sync_copy(bufs[(nch - 1) % 2][1],
                        out_hbm.at[pl.ds(base + (nch - 1) * _CH, _CH)])

    return gather_kernel(table, idx_flat)


def _stage3_body(w2a_ref, w2b_ref, seed_ref, g_ref, b1_ref, b2_ref, out_ref):
    bias = lax.dot_general(
        w2b_ref[...], b1_ref[...], (((1,), (0,)), ((), ())),
        preferred_element_type=jnp.float32) + b2_ref[...]
    a = lax.dot_general(
        w2a_ref[...], seed_ref[0], (((1,), (0,)), ((), ())),
        preferred_element_type=jnp.float32)
    gpart = lax.dot_general(
        w2b_ref[...], g_ref[0], (((1,), (1,)), ((), ())),
        preferred_element_type=jnp.float32)
    out_ref[0] = a + gpart + bias


def _stage3(w2a, w2b, seed_features, g, b1_2d, b2_2d):
    return pl.pallas_call(
        _stage3_body,
        grid=(_B,),
        in_specs=[
            pl.BlockSpec((_D, _D), lambda b: (0, 0)),
            pl.BlockSpec((_D, _D), lambda b: (0, 0)),
            pl.BlockSpec((1, _D, _S), lambda b: (b, 0, 0)),
            pl.BlockSpec((1, _S, _D), lambda b: (b, 0, 0)),
            pl.BlockSpec((_D, 1), lambda b: (0, 0)),
            pl.BlockSpec((_D, 1), lambda b: (0, 0)),
        ],
        out_specs=pl.BlockSpec((1, _D, _S), lambda b: (b, 0, 0)),
        out_shape=jax.ShapeDtypeStruct((_B, _D, _S), jnp.float32),
    )(w2a, w2b, seed_features, g, b1_2d, b2_2d)


def kernel(query_xyz, ref_xyz, ref_feat, seed_features, W1, b1, W2, b2):
    ref_xyz_t = jnp.transpose(ref_xyz, (0, 2, 1))
    w1t = W1.T
    w2a = W2[:, :_D]
    w2b = W2[:, _D:]

    fiota = jnp.arange(_NC, dtype=jnp.float32).reshape(1, _NC)
    gt, idx = _stage1(query_xyz, ref_xyz_t, ref_feat, w1t, fiota)
    g = _sc_gather(gt.reshape(_B * _NC, _D), idx.reshape(_B * _S))
    out = _stage3(w2a, w2b, seed_features, g.reshape(_B, _S, _D),
                  b1.reshape(_D, 1), b2.reshape(_D, 1))
    return out


# speedup vs baseline: 2.5369x; 1.0654x over previous
import functools

import jax
import jax.numpy as jnp
from jax import lax
from jax.experimental import pallas as pl
from jax.experimental.pallas import tpu as pltpu
from jax.experimental.pallas import tpu_sc as plsc

_B, _S, _NC, _C = 8, 2048, 4096, 512
_D = 256
_SBLK = 512
_NCBLK = 1024
_NSTEP = _S // _SBLK

_NWORK = 32
_ROWS_PER_W = _B * _S // _NWORK
_CH = 128


def _stage1_body(q_ref, rt_ref, rf_ref, w1t_ref, fiota_ref, gt_ref, idx_ref):
    b = pl.program_id(0)
    q = q_ref[0]
    rt = rt_ref[0]
    q0, q1, q2 = q[:, 0:1], q[:, 1:2], q[:, 2:3]
    r0, r1, r2 = rt[0:1, :], rt[1:2, :], rt[2:3, :]
    qm2 = (-2.0 * q).astype(jnp.bfloat16)
    rb = rt.astype(jnp.bfloat16)
    acc = lax.dot_general(qm2, rb, (((1,), (0,)), ((), ())),
                          preferred_element_type=jnp.float32)
    qsq = q0 * q0 + q1 * q1 + q2 * q2
    rsq = r0 * r0 + r1 * r1 + r2 * r2
    val = jnp.maximum((qsq + acc[:, 0:128]) + rsq[:, 0:128], 0.0)
    ci = jnp.zeros(val.shape, jnp.float32)
    for c in range(1, _NC // 128):
        lo = c * 128
        dc = jnp.maximum((qsq + acc[:, lo:lo + 128]) + rsq[:, lo:lo + 128], 0.0)
        lt = dc < val
        val = jnp.minimum(val, dc)
        ci = jnp.where(lt, float(lo), ci)
    liota = fiota_ref[:, 0:128]
    idxf = ci + liota
    m = jnp.min(val, axis=1, keepdims=True)
    fidx = jnp.min(jnp.where(val <= m, idxf, float(_NC)), axis=1)
    idx_ref[0, 0, 0, :] = fidx.astype(jnp.int32) + b * _NC
    gt_ref[0] = lax.dot_general(
        rf_ref[0].astype(jnp.bfloat16), w1t_ref[...].astype(jnp.bfloat16),
        (((0,), (0,)), ((), ())),
        preferred_element_type=jnp.float32)


def _stage1(query_xyz, ref_xyz_t, ref_feat, w1t, fiota):
    return pl.pallas_call(
        _stage1_body,
        grid=(_B, _NSTEP),
        in_specs=[
            pl.BlockSpec((1, _SBLK, 3), lambda b, i: (b, i, 0)),
            pl.BlockSpec((1, 3, _NC), lambda b, i: (b, 0, 0)),
            pl.BlockSpec((1, _C, _NCBLK), lambda b, i: (b, 0, i)),
            pl.BlockSpec((_C, _D), lambda b, i: (0, 0)),
            pl.BlockSpec((1, _NC), lambda b, i: (0, 0)),
        ],
        out_specs=[
            pl.BlockSpec((1, _NCBLK, _D), lambda b, i: (b, i, 0)),
            pl.BlockSpec((1, 1, 1, _SBLK), lambda b, i: (b, i, 0, 0)),
        ],
        out_shape=[
            jax.ShapeDtypeStruct((_B, _NC, _D), jnp.float32),
            jax.ShapeDtypeStruct((_B, _NSTEP, 1, _SBLK), jnp.int32),
        ],
    )(query_xyz, ref_xyz_t, ref_feat, w1t, fiota)


def _sc_gather(table, idx_flat):
    mesh = plsc.VectorSubcoreMesh(core_axis_name="c", subcore_axis_name="s")

    @functools.partial(
        pl.kernel,
        mesh=mesh,
        out_type=jax.ShapeDtypeStruct((_B * _S, _D), jnp.float32),
        scratch_types=[
            pltpu.VMEM((_CH,), jnp.int32),
            pltpu.VMEM((_CH,), jnp.int32),
            pltpu.VMEM((_CH, _D), jnp.float32),
            pltpu.VMEM((_CH, _D), jnp.float32),
            pltpu.SemaphoreType.DMA,
            pltpu.SemaphoreType.DMA,
        ],
    )
    def gather_kernel(table_hbm, idx_hbm, out_hbm,
                      idx_v0, idx_v1, rows_v0, rows_v1, sem0, sem1):
        wid = lax.axis_index("s") * 2 + lax.axis_index("c")
        base = wid * _ROWS_PER_W
        bufs = ((idx_v0, rows_v0, sem0), (idx_v1, rows_v1, sem1))
        nch = _ROWS_PER_W // _CH
        copies = [None, None]
        for i in range(nch):
            iv, rv, sm = bufs[i % 2]
            pltpu.sync_copy(idx_hbm.at[pl.ds(base + i * _CH, _CH)], iv)
            copies[i % 2] = pltpu.async_copy(table_hbm.at[iv], rv, sm)
            if i >= 1:
                pv, prv, psm = bufs[(i - 1) % 2]
                copies[(i - 1) % 2].wait()
                pltpu.sync_copy(prv, out_hbm.at[pl.ds(base + (i - 1) * _CH, _CH)])
        copies[(nch - 1) % 2].wait()
        pltpu.sync_copy(bufs[(nch - 1) % 2][1],
                        out_hbm.at[pl.ds(base + (nch - 1) * _CH, _CH)])

    return gather_kernel(table, idx_flat)


def _stage3_body(w2a_ref, w2b_ref, seed_ref, g_ref, b1_ref, b2_ref, out_ref):
    bias = lax.dot_general(
        w2b_ref[...], b1_ref[...], (((1,), (0,)), ((), ())),
        preferred_element_type=jnp.float32) + b2_ref[...]
    a = lax.dot_general(
        w2a_ref[...], seed_ref[0], (((1,), (0,)), ((), ())),
        preferred_element_type=jnp.float32)
    gpart = lax.dot_general(
        w2b_ref[...], g_ref[0], (((1,), (1,)), ((), ())),
        preferred_element_type=jnp.float32)
    out_ref[0] = a + gpart + bias


def _stage3(w2a, w2b, seed_features, g, b1_2d, b2_2d):
    return pl.pallas_call(
        _stage3_body,
        grid=(_B,),
        in_specs=[
            pl.BlockSpec((_D, _D), lambda b: (0, 0)),
            pl.BlockSpec((_D, _D), lambda b: (0, 0)),
            pl.BlockSpec((1, _D, _S), lambda b: (b, 0, 0)),
            pl.BlockSpec((1, _S, _D), lambda b: (b, 0, 0)),
            pl.BlockSpec((_D, 1), lambda b: (0, 0)),
            pl.BlockSpec((_D, 1), lambda b: (0, 0)),
        ],
        out_specs=pl.BlockSpec((1, _D, _S), lambda b: (b, 0, 0)),
        out_shape=jax.ShapeDtypeStruct((_B, _D, _S), jnp.float32),
    )(w2a, w2b, seed_features, g, b1_2d, b2_2d)


def kernel(query_xyz, ref_xyz, ref_feat, seed_features, W1, b1, W2, b2):
    ref_xyz_t = jnp.transpose(ref_xyz, (0, 2, 1))
    w1t = W1.T
    w2a = W2[:, :_D]
    w2b = W2[:, _D:]

    fiota = jnp.arange(_NC, dtype=jnp.float32).reshape(1, _NC)
    gt, idx = _stage1(query_xyz, ref_xyz_t, ref_feat, w1t, fiota)
    g = _sc_gather(gt.reshape(_B * _NC, _D), idx.reshape(_B * _S))
    out = _stage3(w2a, w2b, seed_features, g.reshape(_B, _S, _D),
                  b1.reshape(_D, 1), b2.reshape(_D, 1))
    return out
